# Initial kernel scaffold; baseline (speedup 1.0000x reference)
import jax
import jax.numpy as jnp
from jax.experimental import pallas as pl


def kernel(node_feats, edge_index, etype, basis, w_comp, h_bias, bn_gamma, bn_beta):
    raise NotImplementedError("write your pallas kernel here")


# speedup vs baseline: 2.5717x; 2.5717x over previous
import functools

import jax
import jax.numpy as jnp
from jax import lax
from jax.experimental import pallas as pl
from jax.experimental.pallas import tpu as pltpu
from jax.experimental.pallas import tpu_sc as plsc

N_NODES = 10000
N_EDGES = 320000
F_IN = 128
F_OUT = 128
N_REL = 1344
N_BASES = 4

NC = 2
NS = 16
FH = F_OUT // NC
EPW = N_EDGES // NS
CH = 80
NCHUNKS = EPW // CH
RPT = 624
REM_ROWS = N_NODES - NS * RPT
HBW = N_BASES * FH


def _mm_body(x_ref, w0_ref, w1_ref, o0_ref, o1_ref):
    x = x_ref[...]
    o0_ref[...] = jnp.dot(x, w0_ref[...], preferred_element_type=jnp.float32)
    o1_ref[...] = jnp.dot(x, w1_ref[...], preferred_element_type=jnp.float32)


def _epilogue_body(p_ref, x_ref, b_ref, g_ref, bb_ref, o_ref):
    agg = jnp.concatenate([p_ref[0], p_ref[1]], axis=1)
    h = jnp.maximum(agg + b_ref[...], 0.0) + x_ref[...]
    mean = jnp.mean(h, axis=0, keepdims=True)
    var = jnp.mean((h - mean) ** 2, axis=0, keepdims=True)
    h = g_ref[...] * (h - mean) * lax.rsqrt(var + 1e-5) + bb_ref[...]
    o_ref[...] = jnp.maximum(h, 0.0)


def _sc_edge_kernel(hb0_hbm, hb1_hbm, src_hbm, dst_hbm, etype_hbm, wcomp_hbm,
                    out_hbm, wcomp_v, etype_v, src_v, dst_v, rows_v, msg_v,
                    coeff_v, agg_sh, sem):
    cid = lax.axis_index("c")
    sid = lax.axis_index("s")
    base = sid * EPW

    pltpu.sync_copy(wcomp_hbm, wcomp_v)
    pltpu.sync_copy(etype_hbm.at[pl.ds(base, EPW)], etype_v)

    zero16 = jnp.zeros((16,), jnp.float32)

    def _zrow(r, carry):
        for j in range(FH // 16):
            msg_v[r, pl.ds(j * 16, 16)] = zero16
        return carry

    lax.fori_loop(0, CH, _zrow, 0)
    row0 = sid * RPT
    nfull = RPT // CH
    for k in range(nfull):
        pltpu.sync_copy(msg_v, agg_sh.at[pl.ds(row0 + k * CH, CH)])
    rem = RPT - nfull * CH
    if rem:
        pltpu.sync_copy(msg_v.at[pl.ds(0, rem)],
                        agg_sh.at[pl.ds(row0 + nfull * CH, rem)])

    @pl.when(sid == 0)
    def _zero_tail():
        pltpu.sync_copy(msg_v.at[pl.ds(0, REM_ROWS)],
                        agg_sh.at[pl.ds(NS * RPT, REM_ROWS)])

    plsc.subcore_barrier()

    def _chunk(c, carry):
        off = c * CH
        pltpu.sync_copy(src_hbm.at[pl.ds(base + off, CH)], src_v)
        pltpu.sync_copy(dst_hbm.at[pl.ds(base + off, CH)], dst_v)

        @pl.when(cid == 0)
        def _g0():
            pltpu.async_copy(hb0_hbm.at[src_v], rows_v, sem).wait()

        @pl.when(cid == 1)
        def _g1():
            pltpu.async_copy(hb1_hbm.at[src_v], rows_v, sem).wait()
        for i in range(CH // 16):
            ety = etype_v[pl.ds(off + i * 16, 16)]
            e4 = ety * N_BASES
            for b in range(N_BASES):
                cv = plsc.load_gather(wcomp_v, [e4 + b])
                coeff_v[pl.ds(b * CH + i * 16, 16)] = cv

        def _edge(e, ecarry):
            ev = jnp.zeros((16,), jnp.int32) + e
            c0 = plsc.load_gather(coeff_v, [ev])
            c1 = plsc.load_gather(coeff_v, [ev + CH])
            c2 = plsc.load_gather(coeff_v, [ev + 2 * CH])
            c3 = plsc.load_gather(coeff_v, [ev + 3 * CH])
            for j in range(FH // 16):
                p0, p1 = j * 16, FH + j * 16
                p2, p3 = 2 * FH + j * 16, 3 * FH + j * 16
                acc = c0 * rows_v[e, p0 // 128, pl.ds(p0 % 128, 16)]
                acc = acc + c1 * rows_v[e, p1 // 128, pl.ds(p1 % 128, 16)]
                acc = acc + c2 * rows_v[e, p2 // 128, pl.ds(p2 % 128, 16)]
                acc = acc + c3 * rows_v[e, p3 // 128, pl.ds(p3 % 128, 16)]
                msg_v[e, pl.ds(j * 16, 16)] = acc
            return ecarry

        lax.fori_loop(0, CH, _edge, 0)
        pltpu.sync_copy(msg_v, agg_sh.at[dst_v], add=True)
        return carry

    lax.fori_loop(0, NCHUNKS, _chunk, 0)
    plsc.subcore_barrier()

    pltpu.sync_copy(agg_sh.at[pl.ds(row0, RPT)],
                    out_hbm.at[cid, pl.ds(row0, RPT)])

    @pl.when(sid == 0)
    def _out_tail():
        pltpu.sync_copy(agg_sh.at[pl.ds(NS * RPT, REM_ROWS)],
                        out_hbm.at[cid, pl.ds(NS * RPT, REM_ROWS)])


@functools.partial(
    pl.kernel,
    out_type=jax.ShapeDtypeStruct((NC, N_NODES, FH), jnp.float32),
    mesh=plsc.VectorSubcoreMesh(core_axis_name="c", subcore_axis_name="s"),
    compiler_params=pltpu.CompilerParams(needs_layout_passes=False, use_tc_tiling_on_sc=False),
    scratch_types=[
        pltpu.VMEM((N_REL * N_BASES,), jnp.float32),
        pltpu.VMEM((EPW,), jnp.int32),
        pltpu.VMEM((CH,), jnp.int32),
        pltpu.VMEM((CH,), jnp.int32),
        pltpu.VMEM((CH, HBW // 128, 128), jnp.float32),
        pltpu.VMEM((CH, FH), jnp.float32),
        pltpu.VMEM((N_BASES * CH,), jnp.float32),
        pltpu.VMEM_SHARED((N_NODES, FH), jnp.float32),
        pltpu.SemaphoreType.DMA,
    ],
)
def _sc_aggregate(hb0, hb1, src, dst, etype, wcomp, out, *scratch):
    _sc_edge_kernel(hb0, hb1, src, dst, etype, wcomp, out, *scratch)


def kernel(node_feats, edge_index, etype, basis, w_comp, h_bias, bn_gamma,
           bn_beta):
    src = edge_index[0].astype(jnp.int32)
    dst = edge_index[1].astype(jnp.int32)
    ety = etype.astype(jnp.int32)
    basis_flat2 = basis.reshape(N_BASES, F_IN, NC, FH).transpose(2, 1, 0, 3) \
        .reshape(NC, F_IN, HBW)
    wcomp_flat = w_comp.reshape(-1)

    hb0, hb1 = pl.pallas_call(
        _mm_body,
        out_shape=(
            jax.ShapeDtypeStruct((N_NODES, HBW), jnp.float32),
            jax.ShapeDtypeStruct((N_NODES, HBW), jnp.float32),
        ),
    )(node_feats, basis_flat2[0], basis_flat2[1])

    hb0 = hb0.reshape(N_NODES, HBW // 128, 128)
    hb1 = hb1.reshape(N_NODES, HBW // 128, 128)
    parts = _sc_aggregate(hb0, hb1, src, dst, ety, wcomp_flat)

    out = pl.pallas_call(
        _epilogue_body,
        out_shape=jax.ShapeDtypeStruct((N_NODES, F_OUT), jnp.float32),
    )(parts, node_feats, h_bias.reshape(1, F_OUT),
      bn_gamma.reshape(1, F_OUT), bn_beta.reshape(1, F_OUT))
    return out


# speedup vs baseline: 3.7690x; 1.4655x over previous
import functools

import jax
import jax.numpy as jnp
from jax import lax
from jax.experimental import pallas as pl
from jax.experimental.pallas import tpu as pltpu
from jax.experimental.pallas import tpu_sc as plsc

N_NODES = 10000
N_EDGES = 320000
F_IN = 128
F_OUT = 128
N_REL = 1344
N_BASES = 4

NC = 2
NS = 16
FH = F_OUT // NC
EPW = N_EDGES // NS
CH = 80
NCHUNKS = EPW // CH
BLK = 10
RPT = 624
REM_ROWS = N_NODES - NS * RPT
HBW = N_BASES * FH


def _mm_body(x_ref, w0_ref, w1_ref, o0_ref, o1_ref):
    x = x_ref[...]
    o0_ref[...] = jnp.dot(x, w0_ref[...], preferred_element_type=jnp.float32)
    o1_ref[...] = jnp.dot(x, w1_ref[...], preferred_element_type=jnp.float32)


def _epilogue_body(p_ref, x_ref, b_ref, g_ref, bb_ref, o_ref):
    agg = jnp.concatenate([p_ref[0], p_ref[1]], axis=1)
    h = jnp.maximum(agg + b_ref[...], 0.0) + x_ref[...]
    mean = jnp.mean(h, axis=0, keepdims=True)
    var = jnp.mean((h - mean) ** 2, axis=0, keepdims=True)
    h = g_ref[...] * (h - mean) * lax.rsqrt(var + 1e-5) + bb_ref[...]
    o_ref[...] = jnp.maximum(h, 0.0)


def _sc_edge_kernel(hb0_hbm, hb1_hbm, src_hbm, dst_hbm, etype_hbm, wcomp_hbm,
                    out_hbm, wcomp_v, src_blk, ety_blk, dst_blk, rows_v0,
                    rows_v1, msg_v, coeff_v, agg_sh, gsem0, gsem1):
    cid = lax.axis_index("c")
    sid = lax.axis_index("s")
    base = sid * EPW
    rows_b = (rows_v0, rows_v1)
    gsem = (gsem0, gsem1)

    pltpu.sync_copy(wcomp_hbm, wcomp_v)

    def issue_gather(k, p):
        idx = src_blk.at[pl.ds(k * CH, CH)]

        @pl.when(cid == 0)
        def _g0():
            pltpu.async_copy(hb0_hbm.at[idx], rows_b[p], gsem[p])

        @pl.when(cid == 1)
        def _g1():
            pltpu.async_copy(hb1_hbm.at[idx], rows_b[p], gsem[p])

    def wait_gather(k, p):
        idx = src_blk.at[pl.ds(k * CH, CH)]

        @pl.when(cid == 0)
        def _w0():
            pltpu.make_async_copy(hb0_hbm.at[idx], rows_b[p], gsem[p]).wait()

        @pl.when(cid == 1)
        def _w1():
            pltpu.make_async_copy(hb1_hbm.at[idx], rows_b[p], gsem[p]).wait()

    zero16 = jnp.zeros((16,), jnp.float32)

    def _zrow(r, carry):
        for j in range(FH // 16):
            msg_v[r, pl.ds(j * 16, 16)] = zero16
        return carry

    lax.fori_loop(0, CH, _zrow, 0)
    row0 = sid * RPT
    nfull = RPT // CH
    for k in range(nfull):
        pltpu.sync_copy(msg_v, agg_sh.at[pl.ds(row0 + k * CH, CH)])
    rem = RPT - nfull * CH
    if rem:
        pltpu.sync_copy(msg_v.at[pl.ds(0, rem)],
                        agg_sh.at[pl.ds(row0 + nfull * CH, rem)])

    @pl.when(sid == 0)
    def _zero_tail():
        pltpu.sync_copy(msg_v.at[pl.ds(0, REM_ROWS)],
                        agg_sh.at[pl.ds(NS * RPT, REM_ROWS)])

    plsc.subcore_barrier()

    def _compute_chunk(k, p):
        for i in range(CH // 16):
            ety = ety_blk[pl.ds(k * CH + i * 16, 16)]
            e4 = ety * N_BASES
            for b in range(N_BASES):
                cv = plsc.load_gather(wcomp_v, [e4 + b])
                coeff_v[pl.ds(b * CH + i * 16, 16)] = cv

        rows_v = rows_b[p]

        def _edge(e, ecarry):
            ev = jnp.zeros((16,), jnp.int32) + e
            c0 = plsc.load_gather(coeff_v, [ev])
            c1 = plsc.load_gather(coeff_v, [ev + CH])
            c2 = plsc.load_gather(coeff_v, [ev + 2 * CH])
            c3 = plsc.load_gather(coeff_v, [ev + 3 * CH])
            for j in range(FH // 16):
                p0, p1 = j * 16, FH + j * 16
                p2, p3 = 2 * FH + j * 16, 3 * FH + j * 16
                acc = c0 * rows_v[e, p0 // 128, pl.ds(p0 % 128, 16)]
                acc = acc + c1 * rows_v[e, p1 // 128, pl.ds(p1 % 128, 16)]
                acc = acc + c2 * rows_v[e, p2 // 128, pl.ds(p2 % 128, 16)]
                acc = acc + c3 * rows_v[e, p3 // 128, pl.ds(p3 % 128, 16)]
                msg_v[e, pl.ds(j * 16, 16)] = acc
            return ecarry

        lax.fori_loop(0, CH, _edge, 0, unroll=4)

    def _block(ib, carry):
        boff = ib * (BLK * CH)
        pltpu.sync_copy(src_hbm.at[pl.ds(base + boff, BLK * CH)], src_blk)
        pltpu.sync_copy(etype_hbm.at[pl.ds(base + boff, BLK * CH)], ety_blk)
        pltpu.sync_copy(dst_hbm.at[pl.ds(base + boff, BLK * CH)], dst_blk)
        issue_gather(0, 0)
        for k in range(BLK):
            p = k % 2
            if k + 1 < BLK:
                issue_gather(k + 1, 1 - p)
            wait_gather(k, p)
            _compute_chunk(k, p)
            pltpu.sync_copy(msg_v, agg_sh.at[dst_blk.at[pl.ds(k * CH, CH)]],
                            add=True)
        return carry

    lax.fori_loop(0, NCHUNKS // BLK, _block, 0)
    plsc.subcore_barrier()

    pltpu.sync_copy(agg_sh.at[pl.ds(row0, RPT)],
                    out_hbm.at[cid, pl.ds(row0, RPT)])

    @pl.when(sid == 0)
    def _out_tail():
        pltpu.sync_copy(agg_sh.at[pl.ds(NS * RPT, REM_ROWS)],
                        out_hbm.at[cid, pl.ds(NS * RPT, REM_ROWS)])


@functools.partial(
    pl.kernel,
    out_type=jax.ShapeDtypeStruct((NC, N_NODES, FH), jnp.float32),
    mesh=plsc.VectorSubcoreMesh(core_axis_name="c", subcore_axis_name="s"),
    compiler_params=pltpu.CompilerParams(needs_layout_passes=False, use_tc_tiling_on_sc=False),
    scratch_types=[
        pltpu.VMEM((N_REL * N_BASES,), jnp.float32),
        pltpu.VMEM((BLK * CH,), jnp.int32),
        pltpu.VMEM((BLK * CH,), jnp.int32),
        pltpu.VMEM((BLK * CH,), jnp.int32),
        pltpu.VMEM((CH, HBW // 128, 128), jnp.float32),
        pltpu.VMEM((CH, HBW // 128, 128), jnp.float32),
        pltpu.VMEM((CH, FH), jnp.float32),
        pltpu.VMEM((N_BASES * CH,), jnp.float32),
        pltpu.VMEM_SHARED((N_NODES, FH), jnp.float32),
        pltpu.SemaphoreType.DMA,
        pltpu.SemaphoreType.DMA,
    ],
)
def _sc_aggregate(hb0, hb1, src, dst, etype, wcomp, out, *scratch):
    _sc_edge_kernel(hb0, hb1, src, dst, etype, wcomp, out, *scratch)


def kernel(node_feats, edge_index, etype, basis, w_comp, h_bias, bn_gamma,
           bn_beta):
    src = edge_index[0].astype(jnp.int32)
    dst = edge_index[1].astype(jnp.int32)
    ety = etype.astype(jnp.int32)
    basis_flat2 = basis.reshape(N_BASES, F_IN, NC, FH).transpose(2, 1, 0, 3) \
        .reshape(NC, F_IN, HBW)
    wcomp_flat = w_comp.reshape(-1)

    hb0, hb1 = pl.pallas_call(
        _mm_body,
        out_shape=(
            jax.ShapeDtypeStruct((N_NODES, HBW), jnp.float32),
            jax.ShapeDtypeStruct((N_NODES, HBW), jnp.float32),
        ),
    )(node_feats, basis_flat2[0], basis_flat2[1])

    hb0 = hb0.reshape(N_NODES, HBW // 128, 128)
    hb1 = hb1.reshape(N_NODES, HBW // 128, 128)
    parts = _sc_aggregate(hb0, hb1, src, dst, ety, wcomp_flat)

    out = pl.pallas_call(
        _epilogue_body,
        out_shape=jax.ShapeDtypeStruct((N_NODES, F_OUT), jnp.float32),
    )(parts, node_feats, h_bias.reshape(1, F_OUT),
      bn_gamma.reshape(1, F_OUT), bn_beta.reshape(1, F_OUT))
    return out
